# baseline (speedup 1.0000x reference)
import functools

import jax
import jax.numpy as jnp
from jax import lax
from jax.experimental import pallas as pl
from jax.experimental.pallas import tpu as pltpu
from jax.experimental.pallas import tpu_sc as plsc

_DN = (((1,), (1,)), ((), ()))


def _pack_body(K, vbh, a_ref, b_ref, o_ref):
    at = a_ref[...].T
    bt = b_ref[...].T
    o_ref[...] = jnp.concatenate([at, bt], axis=1)


def _pack_table(tableT, K, vbh):
    E, V = tableT.shape
    nb = K // vbh
    return pl.pallas_call(
        functools.partial(_pack_body, K, vbh),
        grid=(nb,),
        in_specs=[
            pl.BlockSpec((E, vbh), lambda i: (0, i)),
            pl.BlockSpec((E, vbh), lambda i: (0, i + nb)),
        ],
        out_specs=pl.BlockSpec((vbh, 2 * E), lambda i: (i, 0)),
        out_shape=jax.ShapeDtypeStruct((K, 2 * E), jnp.float32),
    )(tableT, tableT)


@functools.lru_cache(maxsize=None)
def _make_sc_gather(K, D2, BPAD):
    info = plsc.get_sparse_core_info()
    nw = info.num_cores * info.num_subcores
    b_per_w = BPAD // nw
    mesh = plsc.VectorSubcoreMesh(core_axis_name="c", subcore_axis_name="s")

    @functools.partial(
        pl.kernel,
        mesh=mesh,
        out_type=jax.ShapeDtypeStruct((BPAD, D2), jnp.float32),
        scratch_types=[
            pltpu.VMEM((b_per_w,), jnp.int32),
            pltpu.VMEM((b_per_w, D2), jnp.float32),
            pltpu.SemaphoreType.DMA,
        ],
    )
    def gather_k(table_hbm, idx_hbm, out_hbm, idx_v, rows_v, sem):
        wid = lax.axis_index("s") * info.num_cores + lax.axis_index("c")
        base = wid * b_per_w
        pltpu.sync_copy(idx_hbm.at[pl.ds(base, b_per_w)], idx_v)
        pltpu.async_copy(table_hbm.at[idx_v], rows_v, sem).wait()
        pltpu.sync_copy(rows_v, out_hbm.at[pl.ds(base, b_per_w)])

    return gather_k


def _lstm_body(T, B, H,
               emb_ref, sub_ref, wih_f_ref, whh_f_ref, bih_f_ref, bhh_f_ref,
               wih_b_ref, whh_b_ref, bih_b_ref, bhh_b_ref,
               wh2l_ref, bh2l_ref, wl2h_ref, bl2h_ref,
               whh_d_ref, bih_d_ref, bhh_d_ref,
               ys_ref, lat_ref):
    f32 = jnp.float32
    emb2 = emb_ref[...]
    E = emb2.shape[1] // 2
    sel = sub_ref[...] > 0
    emb = jnp.where(sel, emb2[:, E:], emb2[:, :E])
    xf = (lax.dot_general(emb, wih_f_ref[...], _DN, preferred_element_type=f32)
          + bih_f_ref[...] + bhh_f_ref[...])
    xb = (lax.dot_general(emb, wih_b_ref[...], _DN, preferred_element_type=f32)
          + bih_b_ref[...] + bhh_b_ref[...])

    def cell(x_t, h, c, whh, nh):
        g = x_t + lax.dot_general(h, whh, _DN, preferred_element_type=f32)
        i = jax.nn.sigmoid(g[:, 0:nh])
        f = jax.nn.sigmoid(g[:, nh:2 * nh])
        gg = jnp.tanh(g[:, 2 * nh:3 * nh])
        o = jax.nn.sigmoid(g[:, 3 * nh:4 * nh])
        c2 = f * c + i * gg
        return o * jnp.tanh(c2), c2

    whh_f = whh_f_ref[...]
    whh_b = whh_b_ref[...]
    h_f = c_f = jnp.zeros((B, H), f32)
    h_b = c_b = jnp.zeros((B, H), f32)
    for t in range(T):
        h_f, c_f = cell(xf[t * B:(t + 1) * B], h_f, c_f, whh_f, H)
        r = T - 1 - t
        h_b, c_b = cell(xb[r * B:(r + 1) * B], h_b, c_b, whh_b, H)

    hcat = jnp.concatenate([h_f, h_b], axis=1)
    lat = (lax.dot_general(hcat, wh2l_ref[...], _DN, preferred_element_type=f32)
           + bh2l_ref[...])
    lat_ref[...] = lat

    h = (lax.dot_general(lat, wl2h_ref[...], _DN, preferred_element_type=f32)
         + bl2h_ref[...])
    c = jnp.zeros((B, 2 * H), f32)
    bd = bih_d_ref[...] + bhh_d_ref[...]
    whh_d = whh_d_ref[...]
    for t in range(T):
        h, c = cell(bd, h, c, whh_d, 2 * H)
        ys_ref[t * B:(t + 1) * B, :] = h


def _proj_body(ys_ref, w_ref, b_ref, o_ref):
    acc = (lax.dot_general(ys_ref[...], w_ref[...], _DN,
                           preferred_element_type=jnp.float32)
           + b_ref[...])
    o_ref[...] = acc.reshape(o_ref.shape)


def kernel(x, table, Wih_fwd, Whh_fwd, bih_fwd, bhh_fwd,
           Wih_bwd, Whh_bwd, bih_bwd, bhh_bwd,
           Wh2l, bh2l, Wl2h, bl2h,
           Wih_dec, Whh_dec, bih_dec, bhh_dec, Wout, bout):
    B, T = x.shape
    V, E = table.shape
    H = Whh_fwd.shape[1]
    L = Wh2l.shape[0]
    n = T * B

    vbh = 3200
    K = -(-((V + 1) // 2) // vbh) * vbh
    packed = _pack_table(table.T, K, vbh)
    bpad = -(-n // 256) * 256
    idx = jnp.zeros((bpad,), jnp.int32).at[:n].set(
        x.T.reshape(-1).astype(jnp.int32))
    idx_pack = jnp.where(idx >= K, idx - K, idx)
    sub = (idx >= K).astype(jnp.int32).reshape(bpad, 1)
    emb2 = _make_sc_gather(K, 2 * E, bpad)(packed, idx_pack)

    r1 = lambda a: a.reshape(1, -1)
    ys, latent = pl.pallas_call(
        functools.partial(_lstm_body, T, B, H),
        in_specs=[pl.BlockSpec(memory_space=pltpu.VMEM)] * 17,
        out_specs=[pl.BlockSpec(memory_space=pltpu.VMEM)] * 2,
        out_shape=[jax.ShapeDtypeStruct((n, 2 * H), jnp.float32),
                   jax.ShapeDtypeStruct((B, L), jnp.float32)],
    )(emb2, sub, Wih_fwd, Whh_fwd, r1(bih_fwd), r1(bhh_fwd),
      Wih_bwd, Whh_bwd, r1(bih_bwd), r1(bhh_bwd),
      Wh2l, r1(bh2l), Wl2h, r1(bl2h),
      Whh_dec, r1(bih_dec), r1(bhh_dec))

    vblk = 6400
    out = pl.pallas_call(
        _proj_body,
        grid=(pl.cdiv(V, vblk),),
        in_specs=[
            pl.BlockSpec((n, 2 * H), lambda i: (0, 0)),
            pl.BlockSpec((vblk, 2 * H), lambda i: (i, 0)),
            pl.BlockSpec((1, vblk), lambda i: (0, i)),
        ],
        out_specs=pl.BlockSpec((T, B, vblk), lambda i: (0, 0, i)),
        out_shape=jax.ShapeDtypeStruct((T, B, V), jnp.float32),
        compiler_params=pltpu.CompilerParams(
            dimension_semantics=("parallel",)),
    )(ys, Wout, r1(bout))

    return jnp.swapaxes(out, 0, 1), latent


# speedup vs baseline: 1.0215x; 1.0215x over previous
import functools

import jax
import jax.numpy as jnp
from jax import lax
from jax.experimental import pallas as pl
from jax.experimental.pallas import tpu as pltpu
from jax.experimental.pallas import tpu_sc as plsc

_DN = (((1,), (1,)), ((), ()))


def _pack_body(K, vbh, a_ref, b_ref, o_ref):
    at = a_ref[...].T
    bt = b_ref[...].T
    o_ref[...] = jnp.concatenate([at, bt], axis=1)


def _pack_table(tableT, K, vbh):
    E, V = tableT.shape
    nb = K // vbh
    return pl.pallas_call(
        functools.partial(_pack_body, K, vbh),
        grid=(nb,),
        in_specs=[
            pl.BlockSpec((E, vbh), lambda i: (0, i)),
            pl.BlockSpec((E, vbh), lambda i: (0, i + nb)),
        ],
        out_specs=pl.BlockSpec((vbh, 2 * E), lambda i: (i, 0)),
        out_shape=jax.ShapeDtypeStruct((K, 2 * E), jnp.float32),
    )(tableT, tableT)


@functools.lru_cache(maxsize=None)
def _make_sc_gather(K, D2, BPAD):
    info = plsc.get_sparse_core_info()
    nw = info.num_cores * info.num_subcores
    b_per_w = BPAD // nw
    mesh = plsc.VectorSubcoreMesh(core_axis_name="c", subcore_axis_name="s")

    @functools.partial(
        pl.kernel,
        mesh=mesh,
        out_type=jax.ShapeDtypeStruct((BPAD, D2), jnp.float32),
        scratch_types=[
            pltpu.VMEM((b_per_w,), jnp.int32),
            pltpu.VMEM((b_per_w, D2), jnp.float32),
            pltpu.SemaphoreType.DMA,
        ],
    )
    def gather_k(table_hbm, idx_hbm, out_hbm, idx_v, rows_v, sem):
        wid = lax.axis_index("s") * info.num_cores + lax.axis_index("c")
        base = wid * b_per_w
        pltpu.sync_copy(idx_hbm.at[pl.ds(base, b_per_w)], idx_v)
        pltpu.async_copy(table_hbm.at[idx_v], rows_v, sem).wait()
        pltpu.sync_copy(rows_v, out_hbm.at[pl.ds(base, b_per_w)])

    return gather_k


def _lstm_body(T, B, H,
               emb_ref, sub_ref, wih_f_ref, whh_f_ref, bih_f_ref, bhh_f_ref,
               wih_b_ref, whh_b_ref, bih_b_ref, bhh_b_ref,
               wh2l_ref, bh2l_ref, wl2h_ref, bl2h_ref,
               whh_d_ref, bih_d_ref, bhh_d_ref,
               ys_ref, lat_ref):
    f32 = jnp.float32
    emb2 = emb_ref[...]
    E = emb2.shape[1] // 2
    sel = sub_ref[...] > 0
    emb = jnp.where(sel, emb2[:, E:], emb2[:, :E])
    xf = (lax.dot_general(emb, wih_f_ref[...], _DN, preferred_element_type=f32)
          + bih_f_ref[...] + bhh_f_ref[...])
    xb = (lax.dot_general(emb, wih_b_ref[...], _DN, preferred_element_type=f32)
          + bih_b_ref[...] + bhh_b_ref[...])

    def cell(x_t, h, c, whh, nh):
        g = x_t + lax.dot_general(h, whh, _DN, preferred_element_type=f32)
        i = jax.nn.sigmoid(g[:, 0:nh])
        f = jax.nn.sigmoid(g[:, nh:2 * nh])
        gg = jnp.tanh(g[:, 2 * nh:3 * nh])
        o = jax.nn.sigmoid(g[:, 3 * nh:4 * nh])
        c2 = f * c + i * gg
        return o * jnp.tanh(c2), c2

    whh_f = whh_f_ref[...]
    whh_b = whh_b_ref[...]
    h_f = c_f = jnp.zeros((B, H), f32)
    h_b = c_b = jnp.zeros((B, H), f32)
    for t in range(T):
        h_f, c_f = cell(xf[t * B:(t + 1) * B], h_f, c_f, whh_f, H)
        r = T - 1 - t
        h_b, c_b = cell(xb[r * B:(r + 1) * B], h_b, c_b, whh_b, H)

    hcat = jnp.concatenate([h_f, h_b], axis=1)
    lat = (lax.dot_general(hcat, wh2l_ref[...], _DN, preferred_element_type=f32)
           + bh2l_ref[...])
    lat_ref[...] = lat

    h = (lax.dot_general(lat, wl2h_ref[...], _DN, preferred_element_type=f32)
         + bl2h_ref[...])
    c = jnp.zeros((B, 2 * H), f32)
    bd = bih_d_ref[...] + bhh_d_ref[...]
    whh_d = whh_d_ref[...]
    for t in range(T):
        h, c = cell(bd, h, c, whh_d, 2 * H)
        ys_ref[t * B:(t + 1) * B, :] = h


def _proj_body(ys_ref, w_ref, b_ref, o_ref):
    acc = (lax.dot_general(ys_ref[...], w_ref[...], _DN,
                           preferred_element_type=jnp.float32)
           + b_ref[...])
    o_ref[...] = acc.reshape(o_ref.shape)


def kernel(x, table, Wih_fwd, Whh_fwd, bih_fwd, bhh_fwd,
           Wih_bwd, Whh_bwd, bih_bwd, bhh_bwd,
           Wh2l, bh2l, Wl2h, bl2h,
           Wih_dec, Whh_dec, bih_dec, bhh_dec, Wout, bout):
    B, T = x.shape
    V, E = table.shape
    H = Whh_fwd.shape[1]
    L = Wh2l.shape[0]
    n = T * B

    vbh = 12800
    K = -(-((V + 1) // 2) // vbh) * vbh
    packed = _pack_table(table.T, K, vbh)
    bpad = -(-n // 256) * 256
    idx = jnp.zeros((bpad,), jnp.int32).at[:n].set(
        x.T.reshape(-1).astype(jnp.int32))
    idx_pack = jnp.where(idx >= K, idx - K, idx)
    sub = (idx >= K).astype(jnp.int32).reshape(bpad, 1)
    emb2 = _make_sc_gather(K, 2 * E, bpad)(packed, idx_pack)

    r1 = lambda a: a.reshape(1, -1)
    ys, latent = pl.pallas_call(
        functools.partial(_lstm_body, T, B, H),
        in_specs=[pl.BlockSpec(memory_space=pltpu.VMEM)] * 17,
        out_specs=[pl.BlockSpec(memory_space=pltpu.VMEM)] * 2,
        out_shape=[jax.ShapeDtypeStruct((n, 2 * H), jnp.float32),
                   jax.ShapeDtypeStruct((B, L), jnp.float32)],
    )(emb2, sub, Wih_fwd, Whh_fwd, r1(bih_fwd), r1(bhh_fwd),
      Wih_bwd, Whh_bwd, r1(bih_bwd), r1(bhh_bwd),
      Wh2l, r1(bh2l), Wl2h, r1(bl2h),
      Whh_dec, r1(bih_dec), r1(bhh_dec))

    vblk = 6400
    out = pl.pallas_call(
        _proj_body,
        grid=(pl.cdiv(V, vblk),),
        in_specs=[
            pl.BlockSpec((n, 2 * H), lambda i: (0, 0)),
            pl.BlockSpec((vblk, 2 * H), lambda i: (i, 0)),
            pl.BlockSpec((1, vblk), lambda i: (0, i)),
        ],
        out_specs=pl.BlockSpec((T, B, vblk), lambda i: (0, 0, i)),
        out_shape=jax.ShapeDtypeStruct((T, B, V), jnp.float32),
        compiler_params=pltpu.CompilerParams(
            dimension_semantics=("parallel",)),
    )(ys, Wout, r1(bout))

    return jnp.swapaxes(out, 0, 1), latent
